# Initial kernel scaffold; baseline (speedup 1.0000x reference)
import jax
import jax.numpy as jnp
from jax.experimental import pallas as pl


def kernel(inp, lengths, table):
    raise NotImplementedError("write your pallas kernel here")


# speedup vs baseline: 1.0936x; 1.0936x over previous
import functools

import jax
import jax.numpy as jnp
from jax import lax
from jax.experimental import pallas as pl
from jax.experimental.pallas import tpu as pltpu
from jax.experimental.pallas import tpu_sc as plsc

SEQ_LEN = 200
BATCH = 4096
EMB_DIM = 32
N = SEQ_LEN * BATCH
ROW = 128
NROWS = N // ROW
NW = 32
ROWS_PER_W = NROWS // NW
CHUNK = 8
NITER = ROWS_PER_W // CHUNK


def _gather_body(idx_hbm, table_hbm, out_hbm, idx_v, rows_v, sem):
    nc = 2
    wid = lax.axis_index("s") * nc + lax.axis_index("c")
    row_base = wid * ROWS_PER_W

    def body(i, _):
        r0 = row_base + i * CHUNK
        pltpu.sync_copy(idx_hbm.at[pl.ds(r0, CHUNK)], idx_v)
        for j in range(CHUNK):
            pltpu.async_copy(
                table_hbm.at[idx_v.at[j]],
                rows_v.at[pl.ds(j * ROW, ROW)],
                sem,
            )
        for j in range(CHUNK):
            pltpu.make_async_copy(
                table_hbm.at[idx_v.at[j]],
                rows_v.at[pl.ds(j * ROW, ROW)],
                sem,
            ).wait()
        pltpu.sync_copy(rows_v, out_hbm.at[pl.ds(r0 * ROW, CHUNK * ROW)])
        return _

    lax.fori_loop(0, NITER, body, None)


@jax.jit
def _emb_lookup(idx2d, table):
    mesh = plsc.VectorSubcoreMesh(core_axis_name="c", subcore_axis_name="s")
    fn = pl.kernel(
        _gather_body,
        out_type=jax.ShapeDtypeStruct((N, EMB_DIM), jnp.float32),
        mesh=mesh,
        scratch_types=[
            pltpu.VMEM((CHUNK, ROW), jnp.int32),
            pltpu.VMEM((CHUNK * ROW, EMB_DIM), jnp.float32),
            pltpu.SemaphoreType.DMA,
        ],
        compiler_params=pltpu.CompilerParams(use_tc_tiling_on_sc=False),
    )
    return fn(idx2d, table)


def kernel(inp, lengths, table):
    idx2d = inp.reshape(NROWS, ROW)
    out = _emb_lookup(idx2d, table)
    return out.reshape(SEQ_LEN, BATCH, EMB_DIM)


# speedup vs baseline: 1.0940x; 1.0003x over previous
import functools

import jax
import jax.numpy as jnp
from jax import lax
from jax.experimental import pallas as pl
from jax.experimental.pallas import tpu as pltpu
from jax.experimental.pallas import tpu_sc as plsc

SEQ_LEN = 200
BATCH = 4096
EMB_DIM = 32
N = SEQ_LEN * BATCH
NW = 32
PER_W = N // NW
CHUNK = 1024
NITER = PER_W // CHUNK


def _gather_body(idx_hbm, table_hbm, out_hbm, idx_v, rows_v, sem):
    nc = 2
    wid = lax.axis_index("s") * nc + lax.axis_index("c")
    base = wid * PER_W

    def body(i, _):
        o0 = base + i * CHUNK
        pltpu.sync_copy(idx_hbm.at[pl.ds(o0, CHUNK)], idx_v)
        pltpu.async_copy(table_hbm.at[idx_v], rows_v, sem).wait()
        pltpu.sync_copy(rows_v, out_hbm.at[pl.ds(o0, CHUNK)])
        return _

    lax.fori_loop(0, NITER, body, None)


@jax.jit
def _emb_lookup(idx, table):
    mesh = plsc.VectorSubcoreMesh(core_axis_name="c", subcore_axis_name="s")
    fn = pl.kernel(
        _gather_body,
        out_type=jax.ShapeDtypeStruct((N, EMB_DIM), jnp.float32),
        mesh=mesh,
        scratch_types=[
            pltpu.VMEM((CHUNK,), jnp.int32),
            pltpu.VMEM((CHUNK, EMB_DIM), jnp.float32),
            pltpu.SemaphoreType.DMA,
        ],
        compiler_params=pltpu.CompilerParams(use_tc_tiling_on_sc=False),
    )
    return fn(idx, table)


def kernel(inp, lengths, table):
    idx = inp.reshape(N)
    out = _emb_lookup(idx, table)
    return out.reshape(SEQ_LEN, BATCH, EMB_DIM)
